# Initial kernel scaffold; baseline (speedup 1.0000x reference)
import jax
import jax.numpy as jnp
from jax.experimental import pallas as pl


def kernel(x, extra_channels, attributes, params):
    raise NotImplementedError("write your pallas kernel here")


# speedup vs baseline: 1.6290x; 1.6290x over previous
import functools

import jax
import jax.numpy as jnp
from jax.experimental import pallas as pl
from jax.experimental.pallas import tpu as pltpu

_SCALE = 2
_H = _W = 64
_S = 72
_ROWS = _H + 2
_NCORE = _ROWS * _S
_MARGIN = 80
_NPAD = _NCORE + 2 * _MARGIN
_CIN = 16
_NF = 64
_CUP = 12


def _conv_acc(src, w_ref, taps=9):
    acc = None
    for t in range(taps):
        dy, dx = t // 3, t % 3
        start = _MARGIN + (dy - 1) * _S + (dx - 1)
        a = src(start)
        p = jnp.dot(a, w_ref[0, t], preferred_element_type=jnp.float32)
        acc = p if acc is None else acc + p
    return acc


def _sft_body(route_ref, xin_ref, wf_ref, wg_ref, wb_ref, wbody_ref, wup_ref,
              bin_ref, bg_ref, bb_ref, bbody_ref, bup_ref,
              out_ref, buf1, buf2):
    b = pl.program_id(0)

    i = jax.lax.broadcasted_iota(jnp.int32, (_NCORE, 1), 0)
    wp = i % _S
    hp = i // _S
    mask = ((wp >= 1) & (wp <= _W) & (hp >= 1) & (hp <= _H)).astype(jnp.float32)

    xin = lambda s: xin_ref[0, pl.ds(s, _NCORE), :]
    f = _conv_acc(xin, wf_ref) + bin_ref[0]
    g = _conv_acc(xin, wg_ref) + bg_ref[0]
    be = _conv_acc(xin, wb_ref) + bb_ref[0]
    f = jnp.maximum(f, 0.0)
    f = (f * (1.0 + g) + be) * mask

    buf1[pl.ds(0, _MARGIN), :] = jnp.zeros((_MARGIN, _NF), jnp.float32)
    buf1[pl.ds(_MARGIN + _NCORE, _MARGIN), :] = jnp.zeros((_MARGIN, _NF), jnp.float32)
    buf1[pl.ds(_MARGIN, _NCORE), :] = f

    f2 = _conv_acc(lambda s: buf1[pl.ds(s, _NCORE), :], wbody_ref) + bbody_ref[0]
    f2 = jnp.maximum(f2, 0.0) * mask

    buf2[pl.ds(0, _MARGIN), :] = jnp.zeros((_MARGIN, _NF), jnp.float32)
    buf2[pl.ds(_MARGIN + _NCORE, _MARGIN), :] = jnp.zeros((_MARGIN, _NF), jnp.float32)
    buf2[pl.ds(_MARGIN, _NCORE), :] = f2

    y = _conv_acc(lambda s: buf2[pl.ds(s, _NCORE), :], wup_ref) + bup_ref[0]
    valid = route_ref[1, b].astype(jnp.float32)
    out_ref[0] = y * valid


def _tap_matrices(w, off):
    cout, cin = w.shape[0], w.shape[1]
    t = jnp.transpose(w, (2, 3, 1, 0)).reshape(9, cin, cout)
    if cin < _CIN:
        t = jnp.pad(t, ((0, 0), (off, _CIN - off - cin), (0, 0)))
    return t


@jax.jit
def kernel(x, extra_channels, attributes, params):
    B = x.shape[0]
    f32 = jnp.float32

    eid = jnp.clip(jnp.floor(attributes), 0.0, 2.0).astype(jnp.int32)
    valid = ((attributes >= 0.0) & (attributes < 3.0)).astype(jnp.int32)
    route = jnp.stack([eid, valid])

    xin = jnp.concatenate([x, extra_channels], axis=1)
    xin = jnp.transpose(xin, (0, 2, 3, 1))
    xin = jnp.pad(xin, ((0, 0), (1, 1), (1, _S - 1 - _W), (0, _CIN - 13)))
    xin = xin.reshape(B, _NCORE, _CIN)
    xin = jnp.pad(xin, ((0, 0), (_MARGIN, _MARGIN), (0, 0)))

    wf = jnp.stack([_tap_matrices(p['W_in'], 0) for p in params])
    wg = jnp.stack([_tap_matrices(p['W_g'], 3) for p in params])
    wb = jnp.stack([_tap_matrices(p['W_b'], 3) for p in params])
    wbody = jnp.stack([_tap_matrices(p['W_body'], 0) for p in params])
    wup = jnp.stack([_tap_matrices(p['W_up'], 0) for p in params])
    bi = jnp.stack([p['b_in'] for p in params])[:, None, :]
    bg = jnp.stack([p['b_g'] for p in params])[:, None, :]
    bb = jnp.stack([p['b_b'] for p in params])[:, None, :]
    bbody = jnp.stack([p['b_body'] for p in params])[:, None, :]
    bu = jnp.stack([p['b_up'] for p in params])[:, None, :]

    def expert_w4(b, r):
        return (r[0, b], 0, 0, 0)

    def expert_b3(b, r):
        return (r[0, b], 0, 0)

    grid_spec = pltpu.PrefetchScalarGridSpec(
        num_scalar_prefetch=1,
        grid=(B,),
        in_specs=[
            pl.BlockSpec((1, _NPAD, _CIN), lambda b, r: (b, 0, 0)),
            pl.BlockSpec((1, 9, _CIN, _NF), expert_w4),
            pl.BlockSpec((1, 9, _CIN, _NF), expert_w4),
            pl.BlockSpec((1, 9, _CIN, _NF), expert_w4),
            pl.BlockSpec((1, 9, _NF, _NF), expert_w4),
            pl.BlockSpec((1, 9, _NF, _CUP), expert_w4),
            pl.BlockSpec((1, 1, _NF), expert_b3),
            pl.BlockSpec((1, 1, _NF), expert_b3),
            pl.BlockSpec((1, 1, _NF), expert_b3),
            pl.BlockSpec((1, 1, _NF), expert_b3),
            pl.BlockSpec((1, 1, _CUP), expert_b3),
        ],
        out_specs=pl.BlockSpec((1, _NCORE, _CUP), lambda b, r: (b, 0, 0)),
        scratch_shapes=[
            pltpu.VMEM((_NPAD, _NF), f32),
            pltpu.VMEM((_NPAD, _NF), f32),
        ],
    )

    y = pl.pallas_call(
        _sft_body,
        grid_spec=grid_spec,
        out_shape=jax.ShapeDtypeStruct((B, _NCORE, _CUP), f32),
    )(route, xin, wf, wg, wb, wbody, wup, bi, bg, bb, bbody, bu)

    y = y.reshape(B, _ROWS, _S, _CUP)[:, 1:1 + _H, 1:1 + _W, :]
    y = y.reshape(B, _H, _W, 3, _SCALE, _SCALE)
    y = jnp.transpose(y, (0, 3, 1, 4, 2, 5))
    return y.reshape(B, 3, _H * _SCALE, _W * _SCALE)
